# baseline (speedup 1.0000x reference)
import functools

import jax
import jax.numpy as jnp
from jax import lax
from jax.experimental import pallas as pl
from jax.experimental.pallas import tpu as pltpu
from jax.experimental.pallas import tpu_sc as plsc

N = 10000
E = 320000
F_IN = 128
F_OUT = 128
K = 3
B = 4
FW = F_IN * B

NC = 2
NS = 16
NW = NC * NS

ROWS_PER_TILE = 320
SR = 32
NSR = 10
TILE_CAP = 10752
SUB_CAP = 1536
CHUNK = 1600
BATCH = 64


def _append(refs, vecs, m, cur):
    c = plsc.cumsum(m.astype(jnp.int32))
    pos = cur + c - 1
    for ref, v in zip(refs, vecs):
        plsc.store_scatter(ref, [pos], v, mask=m)
    return cur + c[15]


def _filter_edges(rows_hbm, cols_hbm, vals_hbm, eb0, eb1,
                  lcols, lvals, lrows, fsem0, fsem1, lo):
    hi = lo + ROWS_PER_TILE
    nblk = E // CHUNK
    srcs = (rows_hbm, cols_hbm, vals_hbm)

    def issue(blk, bufs, sem):
        for src, dst in zip(srcs, bufs):
            pltpu.async_copy(src.at[pl.ds(blk * CHUNK, CHUNK)], dst, sem)

    def drain(bufs, sem):
        for src, dst in zip(srcs, bufs):
            pltpu.make_async_copy(src.at[pl.ds(0, CHUNK)], dst, sem).wait()

    issue(0, eb0, fsem0)

    def blk_body(blk, cur):
        def proc(bufs, sem, obufs, osem):
            def fn(cur):
                drain(bufs, sem)

                @pl.when(blk + 1 < nblk)
                def _():
                    issue(blk + 1, obufs, osem)

                ebuf_r, ebuf_c, ebuf_v = bufs

                def chunk_body(i, cur):
                    r = ebuf_r[pl.ds(i * 16, 16)]
                    m = (r >= lo) & (r < hi)

                    def hit(cur):
                        c = ebuf_c[pl.ds(i * 16, 16)]
                        v = ebuf_v[pl.ds(i * 16, 16)]
                        return _append((lcols, lvals, lrows),
                                       (c, v, r - lo), m, cur)

                    return lax.cond(jnp.any(m), hit, lambda cur: cur, cur)

                return lax.fori_loop(0, CHUNK // 16, chunk_body, cur)

            return fn

        return lax.cond(blk % 2 == 0,
                        proc(eb0, fsem0, eb1, fsem1),
                        proc(eb1, fsem1, eb0, fsem0), cur)

    cnt = lax.fori_loop(0, nblk, blk_body, jnp.int32(0))
    zi = jnp.zeros((16,), jnp.int32)
    lcols[pl.ds(cnt, 16)] = zi
    lvals[pl.ds(cnt, 16)] = jnp.zeros((16,), jnp.float32)
    lrows[pl.ds(cnt, 16)] = zi
    return cnt


def _spmm_ranges(x_hbm, z_hbm, lcols, lvals, lrows, cnt,
                 subc, subv, subr2, stg0, stg1, zbuf, shacc,
                 g0, g1, s0, s1, sid, lo):
    ntrip = (cnt + 15) // 16
    slab = sid * SR

    def sbody(s, _):
        sl = s * SR
        sh = sl + SR
        zf = jnp.zeros((16,), jnp.float32)
        zi = jnp.zeros((16,), jnp.int32)

        def zbody(i, _):
            subc[pl.ds(i * 64, 16)] = zi
            subc[pl.ds(i * 64 + 16, 16)] = zi
            subc[pl.ds(i * 64 + 32, 16)] = zi
            subc[pl.ds(i * 64 + 48, 16)] = zi
            subv[pl.ds(i * 64, 16)] = zf
            subv[pl.ds(i * 64 + 16, 16)] = zf
            subv[pl.ds(i * 64 + 32, 16)] = zf
            subv[pl.ds(i * 64 + 48, 16)] = zf
            for q in range(4):
                subr2[i, pl.ds(q * 16, 16)] = zi
            return 0

        lax.fori_loop(0, SUB_CAP // 64, zbody, 0)

        def cbody(i, scur):
            lr = lrows[pl.ds(i * 16, 16)]
            c = lcols[pl.ds(i * 16, 16)]
            v = lvals[pl.ds(i * 16, 16)]
            m = (lr >= sl) & (lr < sh)
            cs = plsc.cumsum(m.astype(jnp.int32))
            pos = scur + cs - 1
            plsc.store_scatter(subc, [pos], c, mask=m)
            plsc.store_scatter(subv, [pos], v, mask=m)
            plsc.store_scatter(subr2, [pos >> 6, pos & 63],
                               lr - sl + slab, mask=m)
            return scur + cs[15]

        scur = lax.fori_loop(0, ntrip, cbody, jnp.int32(0))

        for q in range(SR // 4):
            pltpu.sync_copy(zbuf, shacc.at[pl.ds(slab + q * 4, 4)])

        nb = (scur + BATCH - 1) // BATCH

        def scale(b, stg):
            for h in range(BATCH // 16):
                vvh = subv[pl.ds(b * BATCH + h * 16, 16)]

                def jbody(j, _):
                    v16 = vvh[jnp.full((16,), j, jnp.int32)]
                    row = h * 16 + j
                    for t in range(FW // 16):
                        stg[row, pl.ds(t * 16, 16)] = (
                            v16 * stg[row, pl.ds(t * 16, 16)])
                    return 0

                lax.fori_loop(0, 16, jbody, 0)

        @pl.when(nb > 0)
        def _():
            pltpu.async_copy(x_hbm.at[subc.at[pl.ds(0, BATCH)]], stg0, g0)

        def bbody(b, _):
            for par, (stg, gsem, ssem, ostg, ogsem, ossem) in enumerate(
                    ((stg0, g0, s0, stg1, g1, s1),
                     (stg1, g1, s1, stg0, g0, s0))):

                @pl.when(b % 2 == par)
                def _():
                    pltpu.make_async_copy(x_hbm.at[pl.ds(0, BATCH)], stg,
                                          gsem).wait()
                    scale(b, stg)

                    @pl.when((b + 1 < nb) & (b >= 1))
                    def _():
                        pltpu.make_async_copy(x_hbm.at[pl.ds(0, BATCH)],
                                              ostg, ossem).wait()

                    @pl.when(b + 1 < nb)
                    def _():
                        idx = subc.at[pl.ds((b + 1) * BATCH, BATCH)]
                        pltpu.async_copy(x_hbm.at[idx], ostg, ogsem)

                    pltpu.async_copy(stg, shacc.at[subr2.at[b]], ssem,
                                     add=True)

            return 0

        lax.fori_loop(0, nb, bbody, 0)

        @pl.when((nb > 1) & (nb % 2 == 0))
        def _():
            pltpu.make_async_copy(x_hbm.at[pl.ds(0, BATCH)], stg0, s0).wait()

        @pl.when((nb > 1) & (nb % 2 == 1))
        def _():
            pltpu.make_async_copy(x_hbm.at[pl.ds(0, BATCH)], stg1, s1).wait()

        @pl.when((nb > 0) & ((nb - 1) % 2 == 0))
        def _():
            pltpu.make_async_copy(x_hbm.at[pl.ds(0, BATCH)], stg0, s0).wait()

        @pl.when((nb > 0) & ((nb - 1) % 2 == 1))
        def _():
            pltpu.make_async_copy(x_hbm.at[pl.ds(0, BATCH)], stg1, s1).wait()

        row0 = lo + sl

        @pl.when(row0 < N)
        def _():
            pltpu.sync_copy(shacc.at[pl.ds(slab, SR)],
                            z_hbm.at[pl.ds(row0, SR)])

        return 0

    lax.fori_loop(0, NSR, sbody, 0)


def _sc_body_a(x_hbm, rows_hbm, cols_hbm, vals_hbm,
               z_hbm, lc_hbm, lv_hbm, lr_hbm, cnt_hbm,
               er0, ec0, ev0, er1, ec1, ev1, lcols, lvals, lrows,
               subc, subv, subr2, stg0, stg1, zbuf, shacc, cntbuf,
               g0, g1, s0, s1, fs0, fs1):
    sid = lax.axis_index("s")
    wid = sid * NC + lax.axis_index("c")
    lo = wid * ROWS_PER_TILE
    _zero_zbuf(zbuf)
    cnt = _filter_edges(rows_hbm, cols_hbm, vals_hbm, (er0, ec0, ev0),
                        (er1, ec1, ev1), lcols, lvals, lrows, fs0, fs1, lo)
    pltpu.sync_copy(lcols, lc_hbm.at[wid])
    pltpu.sync_copy(lvals, lv_hbm.at[wid])
    pltpu.sync_copy(lrows, lr_hbm.at[wid])
    cntbuf[...] = jnp.full((16,), cnt, jnp.int32)
    pltpu.sync_copy(cntbuf, cnt_hbm.at[wid])
    _spmm_ranges(x_hbm, z_hbm, lcols, lvals, lrows, cnt,
                 subc, subv, subr2, stg0, stg1, zbuf, shacc,
                 g0, g1, s0, s1, sid, lo)


def _sc_body_b(x_hbm, lc_hbm, lv_hbm, lr_hbm, cnt_hbm,
               z_hbm,
               lcols, lvals, lrows, subc, subv, subr2, stg0, stg1, zbuf,
               shacc, cntbuf, g0, g1, s0, s1):
    sid = lax.axis_index("s")
    wid = sid * NC + lax.axis_index("c")
    lo = wid * ROWS_PER_TILE
    _zero_zbuf(zbuf)
    pltpu.sync_copy(lc_hbm.at[wid], lcols)
    pltpu.sync_copy(lv_hbm.at[wid], lvals)
    pltpu.sync_copy(lr_hbm.at[wid], lrows)
    pltpu.sync_copy(cnt_hbm.at[wid], cntbuf)
    cnt = cntbuf[pl.ds(0, 16)][0]
    _spmm_ranges(x_hbm, z_hbm, lcols, lvals, lrows, cnt,
                 subc, subv, subr2, stg0, stg1, zbuf, shacc,
                 g0, g1, s0, s1, sid, lo)


def _tc_matmul_body(x0_ref, z1_ref, z2_ref, w_ref, o_ref):
    a = jnp.dot(x0_ref[...], w_ref[0], preferred_element_type=jnp.float32)
    b = jnp.dot(z1_ref[...], w_ref[1], preferred_element_type=jnp.float32)
    c = jnp.dot(z2_ref[...], w_ref[2], preferred_element_type=jnp.float32)
    o_ref[...] = a + b + c


def _zero_zbuf(zbuf):
    zrow = jnp.zeros((16,), jnp.float32)

    def zb(r, _):
        for t in range(FW // 16):
            zbuf[r, pl.ds(t * 16, 16)] = zrow
        return 0

    lax.fori_loop(0, 4, zb, 0)


_SPMM_SCRATCH = [
    pltpu.VMEM((TILE_CAP,), jnp.int32),
    pltpu.VMEM((TILE_CAP,), jnp.float32),
    pltpu.VMEM((TILE_CAP,), jnp.int32),
    pltpu.VMEM((SUB_CAP,), jnp.int32),
    pltpu.VMEM((SUB_CAP,), jnp.float32),
    pltpu.VMEM((SUB_CAP // BATCH, BATCH), jnp.int32),
    pltpu.VMEM((BATCH, FW), jnp.float32),
    pltpu.VMEM((BATCH, FW), jnp.float32),
    pltpu.VMEM((4, FW), jnp.float32),
    pltpu.VMEM_SHARED((NS * SR, FW), jnp.float32),
    pltpu.VMEM((16,), jnp.int32),
    pltpu.SemaphoreType.DMA,
    pltpu.SemaphoreType.DMA,
    pltpu.SemaphoreType.DMA,
    pltpu.SemaphoreType.DMA,
]


def kernel(input_tensor, w, l_rows, l_cols, l_vals):
    x0 = jnp.transpose(input_tensor, (1, 2, 0)).reshape(N, FW)

    mesh = plsc.VectorSubcoreMesh(core_axis_name="c", subcore_axis_name="s")
    sc_params = pltpu.CompilerParams(use_tc_tiling_on_sc=False,
                                     needs_layout_passes=False)

    phase_a = pl.kernel(
        _sc_body_a,
        out_type=(
            jax.ShapeDtypeStruct((N, FW), jnp.float32),
            jax.ShapeDtypeStruct((NW, TILE_CAP), jnp.int32),
            jax.ShapeDtypeStruct((NW, TILE_CAP), jnp.float32),
            jax.ShapeDtypeStruct((NW, TILE_CAP), jnp.int32),
            jax.ShapeDtypeStruct((NW, 16), jnp.int32),
        ),
        mesh=mesh,
        scratch_types=[
            pltpu.VMEM((CHUNK,), jnp.int32),
            pltpu.VMEM((CHUNK,), jnp.int32),
            pltpu.VMEM((CHUNK,), jnp.float32),
            pltpu.VMEM((CHUNK,), jnp.int32),
            pltpu.VMEM((CHUNK,), jnp.int32),
            pltpu.VMEM((CHUNK,), jnp.float32),
        ] + _SPMM_SCRATCH + [pltpu.SemaphoreType.DMA,
                             pltpu.SemaphoreType.DMA],
        compiler_params=sc_params,
    )
    z1, lc, lv, lr, cnt = phase_a(x0, l_rows, l_cols, l_vals)

    phase_b = pl.kernel(
        _sc_body_b,
        out_type=jax.ShapeDtypeStruct((N, FW), jnp.float32),
        mesh=mesh,
        scratch_types=_SPMM_SCRATCH,
        compiler_params=sc_params,
    )
    z2 = phase_b(z1, lc, lv, lr, cnt)

    wk = w.reshape(F_IN, K, F_OUT)
    v0 = wk[:, 0, :] - wk[:, 2, :]
    v1 = wk[:, 1, :]
    v2 = 2.0 * wk[:, 2, :]
    eye = jnp.eye(B, dtype=jnp.float32)
    we = jnp.stack([jnp.kron(v0, eye), jnp.kron(v1, eye), jnp.kron(v2, eye)])

    BM = 400
    yp = pl.pallas_call(
        _tc_matmul_body,
        grid=(N // BM,),
        in_specs=[
            pl.BlockSpec((BM, FW), lambda i: (i, 0)),
            pl.BlockSpec((BM, FW), lambda i: (i, 0)),
            pl.BlockSpec((BM, FW), lambda i: (i, 0)),
            pl.BlockSpec((K, FW, F_OUT * B), lambda i: (0, 0, 0)),
        ],
        out_specs=pl.BlockSpec((BM, F_OUT * B), lambda i: (i, 0)),
        out_shape=jax.ShapeDtypeStruct((N, F_OUT * B), jnp.float32),
    )(x0, z1, z2, we)

    return jnp.transpose(yp.reshape(N, F_OUT, B), (2, 0, 1))


# speedup vs baseline: 1.3881x; 1.3881x over previous
import functools

import jax
import jax.numpy as jnp
from jax import lax
from jax.experimental import pallas as pl
from jax.experimental.pallas import tpu as pltpu
from jax.experimental.pallas import tpu_sc as plsc

N = 10000
E = 320000
F_IN = 128
F_OUT = 128
K = 3
B = 4
FW = F_IN * B

NC = 2
NS = 16
NW = NC * NS

ROWS_PER_TILE = 320
SR = 80
NSR = 4
TILE_CAP = 11264
SUB_CAP = 3072
CHUNK = 1600
BATCH = 32


def _append(refs, vecs, m, cur):
    c = plsc.cumsum(m.astype(jnp.int32))
    pos = cur + c - 1
    for ref, v in zip(refs, vecs):
        plsc.store_scatter(ref, [pos], v, mask=m)
    return cur + c[15]


def _filter_edges(rows_hbm, cols_hbm, vals_hbm, eb0, eb1,
                  lcols, lvals, lrows, fsem0, fsem1, lo):
    hi = lo + ROWS_PER_TILE
    nblk = E // CHUNK
    srcs = (rows_hbm, cols_hbm, vals_hbm)

    def issue(blk, bufs, sem):
        for src, dst in zip(srcs, bufs):
            pltpu.async_copy(src.at[pl.ds(blk * CHUNK, CHUNK)], dst, sem)

    def drain(bufs, sem):
        for src, dst in zip(srcs, bufs):
            pltpu.make_async_copy(src.at[pl.ds(0, CHUNK)], dst, sem).wait()

    issue(0, eb0, fsem0)

    def blk_body(blk, cur):
        def proc(bufs, sem, obufs, osem):
            def fn(cur):
                drain(bufs, sem)

                @pl.when(blk + 1 < nblk)
                def _():
                    issue(blk + 1, obufs, osem)

                ebuf_r, ebuf_c, ebuf_v = bufs

                def chunk_body(i, cur):
                    r = ebuf_r[pl.ds(i * 16, 16)]
                    m = (r >= lo) & (r < hi)

                    def hit(cur):
                        c = ebuf_c[pl.ds(i * 16, 16)]
                        v = ebuf_v[pl.ds(i * 16, 16)]
                        return _append((lcols, lvals, lrows),
                                       (c, v, r - lo), m, cur)

                    return lax.cond(jnp.any(m), hit, lambda cur: cur, cur)

                return lax.fori_loop(0, CHUNK // 16, chunk_body, cur)

            return fn

        return lax.cond(blk % 2 == 0,
                        proc(eb0, fsem0, eb1, fsem1),
                        proc(eb1, fsem1, eb0, fsem0), cur)

    cnt = lax.fori_loop(0, nblk, blk_body, jnp.int32(0))
    zi = jnp.zeros((16,), jnp.int32)
    lcols[pl.ds(cnt, 16)] = zi
    lvals[pl.ds(cnt, 16)] = jnp.zeros((16,), jnp.float32)
    lrows[pl.ds(cnt, 16)] = zi
    return cnt


def _spmm_ranges(x_hbm, z_hbm, lcols, lvals, lrows, cnt,
                 subc, subv, subr2, stg0, stg1, zbuf, shacc,
                 g0, g1, s0, s1, sid, lo):
    ntrip = (cnt + 15) // 16
    slab = sid * SR

    def sbody(s, _):
        sl = s * SR
        sh = sl + SR
        zf = jnp.zeros((16,), jnp.float32)
        zi = jnp.zeros((16,), jnp.int32)

        def zbody(i, _):
            subc[pl.ds(i * 32, 16)] = zi
            subc[pl.ds(i * 32 + 16, 16)] = zi
            subv[pl.ds(i * 32, 16)] = zf
            subv[pl.ds(i * 32 + 16, 16)] = zf
            subr2[i, pl.ds(0, 16)] = zi
            subr2[i, pl.ds(16, 16)] = zi
            return 0

        lax.fori_loop(0, SUB_CAP // 32, zbody, 0)

        def cbody(i, scur):
            lr = lrows[pl.ds(i * 16, 16)]
            c = lcols[pl.ds(i * 16, 16)]
            v = lvals[pl.ds(i * 16, 16)]
            m = (lr >= sl) & (lr < sh)
            cs = plsc.cumsum(m.astype(jnp.int32))
            pos = scur + cs - 1
            plsc.store_scatter(subc, [pos], c, mask=m)
            plsc.store_scatter(subv, [pos], v, mask=m)
            plsc.store_scatter(subr2, [pos >> 5, pos & 31],
                               lr - sl + slab, mask=m)
            return scur + cs[15]

        scur = lax.fori_loop(0, ntrip, cbody, jnp.int32(0))

        for q in range(SR // 8):
            pltpu.sync_copy(zbuf, shacc.at[pl.ds(slab + q * 8, 8)])

        nb = (scur + BATCH - 1) // BATCH

        def scale(b, stg):
            for h in range(BATCH // 16):
                vvh = subv[pl.ds(b * BATCH + h * 16, 16)]

                def jbody(j, _):
                    v16 = vvh[jnp.full((16,), j, jnp.int32)]
                    row = h * 16 + j
                    for t in range(FW // 16):
                        stg[row, pl.ds(t * 16, 16)] = (
                            v16 * stg[row, pl.ds(t * 16, 16)])
                    return 0

                lax.fori_loop(0, 16, jbody, 0)

        @pl.when(nb > 0)
        def _():
            pltpu.async_copy(x_hbm.at[subc.at[pl.ds(0, BATCH)]], stg0, g0)

        def bbody(b, _):
            for par, (stg, gsem, ssem, ostg, ogsem, ossem) in enumerate(
                    ((stg0, g0, s0, stg1, g1, s1),
                     (stg1, g1, s1, stg0, g0, s0))):

                @pl.when(b % 2 == par)
                def _():
                    pltpu.make_async_copy(x_hbm.at[pl.ds(0, BATCH)], stg,
                                          gsem).wait()
                    scale(b, stg)

                    @pl.when((b + 1 < nb) & (b >= 1))
                    def _():
                        pltpu.make_async_copy(x_hbm.at[pl.ds(0, BATCH)],
                                              ostg, ossem).wait()

                    @pl.when(b + 1 < nb)
                    def _():
                        idx = subc.at[pl.ds((b + 1) * BATCH, BATCH)]
                        pltpu.async_copy(x_hbm.at[idx], ostg, ogsem)

                    pltpu.async_copy(stg, shacc.at[subr2.at[b]], ssem,
                                     add=True)

            return 0

        lax.fori_loop(0, nb, bbody, 0)

        @pl.when((nb > 1) & (nb % 2 == 0))
        def _():
            pltpu.make_async_copy(x_hbm.at[pl.ds(0, BATCH)], stg0, s0).wait()

        @pl.when((nb > 1) & (nb % 2 == 1))
        def _():
            pltpu.make_async_copy(x_hbm.at[pl.ds(0, BATCH)], stg1, s1).wait()

        @pl.when((nb > 0) & ((nb - 1) % 2 == 0))
        def _():
            pltpu.make_async_copy(x_hbm.at[pl.ds(0, BATCH)], stg0, s0).wait()

        @pl.when((nb > 0) & ((nb - 1) % 2 == 1))
        def _():
            pltpu.make_async_copy(x_hbm.at[pl.ds(0, BATCH)], stg1, s1).wait()

        row0 = lo + sl

        @pl.when(row0 < N)
        def _():
            pltpu.sync_copy(shacc.at[pl.ds(slab, SR)],
                            z_hbm.at[pl.ds(row0, SR)])

        return 0

    lax.fori_loop(0, NSR, sbody, 0)


def _sc_body_a(x_hbm, rows_hbm, cols_hbm, vals_hbm,
               z_hbm, lc_hbm, lv_hbm, lr_hbm, cnt_hbm,
               er0, ec0, ev0, er1, ec1, ev1, lcols, lvals, lrows,
               subc, subv, subr2, stg0, stg1, zbuf, shacc, cntbuf,
               g0, g1, s0, s1, fs0, fs1):
    sid = lax.axis_index("s")
    wid = sid * NC + lax.axis_index("c")
    lo = wid * ROWS_PER_TILE
    _zero_zbuf(zbuf)
    cnt = _filter_edges(rows_hbm, cols_hbm, vals_hbm, (er0, ec0, ev0),
                        (er1, ec1, ev1), lcols, lvals, lrows, fs0, fs1, lo)
    pltpu.sync_copy(lcols, lc_hbm.at[wid])
    pltpu.sync_copy(lvals, lv_hbm.at[wid])
    pltpu.sync_copy(lrows, lr_hbm.at[wid])
    cntbuf[...] = jnp.full((16,), cnt, jnp.int32)
    pltpu.sync_copy(cntbuf, cnt_hbm.at[wid])
    _spmm_ranges(x_hbm, z_hbm, lcols, lvals, lrows, cnt,
                 subc, subv, subr2, stg0, stg1, zbuf, shacc,
                 g0, g1, s0, s1, sid, lo)


def _sc_body_b(x_hbm, lc_hbm, lv_hbm, lr_hbm, cnt_hbm,
               z_hbm,
               lcols, lvals, lrows, subc, subv, subr2, stg0, stg1, zbuf,
               shacc, cntbuf, g0, g1, s0, s1):
    sid = lax.axis_index("s")
    wid = sid * NC + lax.axis_index("c")
    lo = wid * ROWS_PER_TILE
    _zero_zbuf(zbuf)
    pltpu.sync_copy(lc_hbm.at[wid], lcols)
    pltpu.sync_copy(lv_hbm.at[wid], lvals)
    pltpu.sync_copy(lr_hbm.at[wid], lrows)
    pltpu.sync_copy(cnt_hbm.at[wid], cntbuf)
    cnt = cntbuf[pl.ds(0, 16)][0]
    _spmm_ranges(x_hbm, z_hbm, lcols, lvals, lrows, cnt,
                 subc, subv, subr2, stg0, stg1, zbuf, shacc,
                 g0, g1, s0, s1, sid, lo)


def _tc_matmul_body(x0_ref, z1_ref, z2_ref, w_ref, o_ref):
    a = jnp.dot(x0_ref[...], w_ref[0], preferred_element_type=jnp.float32)
    b = jnp.dot(z1_ref[...], w_ref[1], preferred_element_type=jnp.float32)
    c = jnp.dot(z2_ref[...], w_ref[2], preferred_element_type=jnp.float32)
    o_ref[...] = a + b + c


def _zero_zbuf(zbuf):
    zrow = jnp.zeros((16,), jnp.float32)

    def zb(r, _):
        for t in range(FW // 16):
            zbuf[r, pl.ds(t * 16, 16)] = zrow
        return 0

    lax.fori_loop(0, 8, zb, 0)


_SPMM_SCRATCH = [
    pltpu.VMEM((TILE_CAP,), jnp.int32),
    pltpu.VMEM((TILE_CAP,), jnp.float32),
    pltpu.VMEM((TILE_CAP,), jnp.int32),
    pltpu.VMEM((SUB_CAP,), jnp.int32),
    pltpu.VMEM((SUB_CAP,), jnp.float32),
    pltpu.VMEM((SUB_CAP // BATCH, BATCH), jnp.int32),
    pltpu.VMEM((BATCH, FW), jnp.float32),
    pltpu.VMEM((BATCH, FW), jnp.float32),
    pltpu.VMEM((8, FW), jnp.float32),
    pltpu.VMEM_SHARED((NS * SR, FW), jnp.float32),
    pltpu.VMEM((16,), jnp.int32),
    pltpu.SemaphoreType.DMA,
    pltpu.SemaphoreType.DMA,
    pltpu.SemaphoreType.DMA,
    pltpu.SemaphoreType.DMA,
]


def kernel(input_tensor, w, l_rows, l_cols, l_vals):
    x0 = jnp.transpose(input_tensor, (1, 2, 0)).reshape(N, FW)

    mesh = plsc.VectorSubcoreMesh(core_axis_name="c", subcore_axis_name="s")
    sc_params = pltpu.CompilerParams(use_tc_tiling_on_sc=False,
                                     needs_layout_passes=False)

    phase_a = pl.kernel(
        _sc_body_a,
        out_type=(
            jax.ShapeDtypeStruct((N, FW), jnp.float32),
            jax.ShapeDtypeStruct((NW, TILE_CAP), jnp.int32),
            jax.ShapeDtypeStruct((NW, TILE_CAP), jnp.float32),
            jax.ShapeDtypeStruct((NW, TILE_CAP), jnp.int32),
            jax.ShapeDtypeStruct((NW, 16), jnp.int32),
        ),
        mesh=mesh,
        scratch_types=[
            pltpu.VMEM((CHUNK,), jnp.int32),
            pltpu.VMEM((CHUNK,), jnp.int32),
            pltpu.VMEM((CHUNK,), jnp.float32),
            pltpu.VMEM((CHUNK,), jnp.int32),
            pltpu.VMEM((CHUNK,), jnp.int32),
            pltpu.VMEM((CHUNK,), jnp.float32),
        ] + _SPMM_SCRATCH + [pltpu.SemaphoreType.DMA,
                             pltpu.SemaphoreType.DMA],
        compiler_params=sc_params,
    )
    z1, lc, lv, lr, cnt = phase_a(x0, l_rows, l_cols, l_vals)

    phase_b = pl.kernel(
        _sc_body_b,
        out_type=jax.ShapeDtypeStruct((N, FW), jnp.float32),
        mesh=mesh,
        scratch_types=_SPMM_SCRATCH,
        compiler_params=sc_params,
    )
    z2 = phase_b(z1, lc, lv, lr, cnt)

    wk = w.reshape(F_IN, K, F_OUT)
    v0 = wk[:, 0, :] - wk[:, 2, :]
    v1 = wk[:, 1, :]
    v2 = 2.0 * wk[:, 2, :]
    eye = jnp.eye(B, dtype=jnp.float32)
    we = jnp.stack([jnp.kron(v0, eye), jnp.kron(v1, eye), jnp.kron(v2, eye)])

    BM = 400
    yp = pl.pallas_call(
        _tc_matmul_body,
        grid=(N // BM,),
        in_specs=[
            pl.BlockSpec((BM, FW), lambda i: (i, 0)),
            pl.BlockSpec((BM, FW), lambda i: (i, 0)),
            pl.BlockSpec((BM, FW), lambda i: (i, 0)),
            pl.BlockSpec((K, FW, F_OUT * B), lambda i: (0, 0, 0)),
        ],
        out_specs=pl.BlockSpec((BM, F_OUT * B), lambda i: (i, 0)),
        out_shape=jax.ShapeDtypeStruct((N, F_OUT * B), jnp.float32),
    )(x0, z1, z2, we)

    return jnp.transpose(yp.reshape(N, F_OUT, B), (2, 0, 1))
